# baseline (speedup 1.0000x reference)
import functools

import jax
import jax.numpy as jnp
from jax.experimental import pallas as pl

EDGE_LENGTH = 0.01
FILTER_K = 32
CENTER_N = 4096
OUT_CH = 32
HIDDEN = 16

CB = 256


def _fc_kernel(pts_ref, ctr_ref, W1_ref, b1_ref, W2_ref, b2_ref, bias_ref,
               out_ref, *, n_points, k_sel):
    pts = pts_ref[0]
    coords = pts[:, :3]
    centers = ctr_ref[0]
    ccoords = centers[:, :3]

    c2 = jnp.sum(ccoords * ccoords, axis=1, keepdims=True)
    p2 = jnp.sum(coords * coords, axis=1)[None, :]
    dot = jax.lax.dot_general(
        ccoords, coords, (((1,), (1,)), ((), ())),
        preferred_element_type=jnp.float32)
    d2 = c2 + p2 - 2.0 * dot

    iota = jax.lax.broadcasted_iota(jnp.int32, d2.shape, 1)
    W1 = W1_ref[...]
    b1 = b1_ref[0]
    W2 = W2_ref[...]
    b2 = b2_ref[0]

    def body(_, carry):
        m, i, acc = carry
        live = (d2 > m) | ((d2 == m) & (iota > i))
        dm = jnp.where(live, d2, jnp.inf)
        m2 = jnp.min(dm, axis=1, keepdims=True)
        i2 = jnp.min(jnp.where(dm == m2, iota, n_points),
                     axis=1, keepdims=True)
        onehot = (iota == i2).astype(jnp.float32)
        sel = jax.lax.dot_general(
            onehot, pts, (((1,), (0,)), ((), ())),
            precision=jax.lax.Precision.HIGHEST,
            preferred_element_type=jnp.float32)
        rel = (sel[:, :3] - ccoords) / EDGE_LENGTH
        h = jax.nn.relu(
            jax.lax.dot_general(rel.astype(jnp.bfloat16),
                                W1.astype(jnp.bfloat16),
                                (((1,), (0,)), ((), ())),
                                preferred_element_type=jnp.float32) + b1)
        w = jax.lax.dot_general(
            h.astype(jnp.bfloat16), W2.astype(jnp.bfloat16),
            (((1,), (0,)), ((), ())),
            preferred_element_type=jnp.float32) + b2
        acc = acc + sel[:, 3:4] * w
        return m2, i2, acc

    m0 = jnp.full((centers.shape[0], 1), -jnp.inf, dtype=jnp.float32)
    i0 = jnp.full((centers.shape[0], 1), -1, dtype=jnp.int32)
    acc0 = jnp.zeros((centers.shape[0], OUT_CH), dtype=jnp.float32)
    _, _, acc = jax.lax.fori_loop(0, k_sel, body, (m0, i0, acc0))

    out_ref[0] = acc / k_sel + bias_ref[0]


def kernel(points_sdf, W1, b1, W2, b2, bias):
    B, N, _ = points_sdf.shape
    nblk = CENTER_N // CB

    fn = functools.partial(_fc_kernel, n_points=N, k_sel=FILTER_K)
    out = pl.pallas_call(
        fn,
        grid=(B, nblk),
        in_specs=[
            pl.BlockSpec((1, N, 4), lambda b, j: (b, 0, 0)),
            pl.BlockSpec((1, CB, 4), lambda b, j: (b, j, 0)),
            pl.BlockSpec((3, HIDDEN), lambda b, j: (0, 0)),
            pl.BlockSpec((1, HIDDEN), lambda b, j: (0, 0)),
            pl.BlockSpec((HIDDEN, OUT_CH), lambda b, j: (0, 0)),
            pl.BlockSpec((1, OUT_CH), lambda b, j: (0, 0)),
            pl.BlockSpec((1, OUT_CH), lambda b, j: (0, 0)),
        ],
        out_specs=pl.BlockSpec((1, CB, OUT_CH), lambda b, j: (b, j, 0)),
        out_shape=jax.ShapeDtypeStruct((B, CENTER_N, OUT_CH), jnp.float32),
    )(points_sdf, points_sdf[:, :CENTER_N, :], W1, b1[None, :], W2,
      b2[None, :], bias[None, :])
    return out


# speedup vs baseline: 2.8093x; 2.8093x over previous
import functools

import jax
import jax.numpy as jnp
from jax import lax
from jax.experimental import pallas as pl
from jax.experimental.pallas import tpu as pltpu
from jax.experimental.pallas import tpu_sc as plsc

EDGE_LENGTH = 0.01
FILTER_K = 32
CENTER_N = 4096
OUT_CH = 32
HIDDEN = 16
ROW_D = 4

CB = 256
CB2 = 512


def _select_kernel(pts_ref, ctr_ref, idx_ref, *, n_points, k_sel):
    pts = pts_ref[0]
    coords = pts[:, :3]
    ccoords = ctr_ref[0][:, :3]
    nrows = ccoords.shape[0]

    c2 = jnp.sum(ccoords * ccoords, axis=1, keepdims=True)
    p2 = jnp.sum(coords * coords, axis=1)[None, :]
    dot = lax.dot_general(
        ccoords, coords, (((1,), (1,)), ((), ())),
        preferred_element_type=jnp.float32)
    d2 = c2 + p2 - 2.0 * dot

    iota = lax.broadcasted_iota(jnp.int32, d2.shape, 1)
    lane = lax.broadcasted_iota(jnp.int32, (nrows, k_sel * ROW_D), 1)
    base = pl.program_id(0) * n_points

    def body(k, carry):
        m, i, idxs = carry
        live = (d2 > m) | ((d2 == m) & (iota > i))
        dm = jnp.where(live, d2, jnp.inf)
        m2 = jnp.min(dm, axis=1, keepdims=True)
        i2 = jnp.min(jnp.where(dm == m2, iota, n_points),
                     axis=1, keepdims=True)
        elem = (i2 + base) * ROW_D + (lane & (ROW_D - 1))
        idxs = jnp.where((lane // ROW_D) == k, elem, idxs)
        return m2, i2, idxs

    m0 = jnp.full((nrows, 1), -jnp.inf, dtype=jnp.float32)
    i0 = jnp.full((nrows, 1), -1, dtype=jnp.int32)
    idx0 = jnp.zeros((nrows, k_sel * ROW_D), dtype=jnp.int32)
    _, _, idxs = lax.fori_loop(0, k_sel, body, (m0, i0, idx0))
    idx_ref[0] = idxs


def _mlp_kernel(nbr_ref, ctr_ref, W1_ref, b1_ref, W2_ref, b2_ref, bias_ref,
                out_ref, *, k_sel):
    nrows = ctr_ref.shape[1]
    nbr = nbr_ref[0]
    ctr = ctr_ref[0][:, :3]

    nbr3 = nbr[:, :3].reshape(nrows, k_sel, 3)
    rel = (nbr3 - ctr[:, None, :]) / EDGE_LENGTH
    rel = rel.reshape(nrows * k_sel, 3)
    h = jax.nn.relu(
        lax.dot_general(rel.astype(jnp.bfloat16),
                        W1_ref[...].astype(jnp.bfloat16),
                        (((1,), (0,)), ((), ())),
                        preferred_element_type=jnp.float32) + b1_ref[0])
    w = lax.dot_general(
        h.astype(jnp.bfloat16), W2_ref[...].astype(jnp.bfloat16),
        (((1,), (0,)), ((), ())),
        preferred_element_type=jnp.float32) + b2_ref[0]
    w = w * nbr[:, 3:4]
    acc = jnp.sum(w.reshape(nrows, k_sel, OUT_CH), axis=1)
    out_ref[0] = acc / k_sel + bias_ref[0]


def _sc_gather(table_flat, idx_flat):
    info = plsc.get_sparse_core_info()
    nw = info.num_cores * info.num_subcores
    btot = idx_flat.shape[0]
    b_per_w = btot // nw
    chunk = min(b_per_w, 4096)
    nchunk = b_per_w // chunk
    mesh = plsc.VectorSubcoreMesh(core_axis_name="c", subcore_axis_name="s")

    @functools.partial(
        pl.kernel, mesh=mesh,
        out_type=jax.ShapeDtypeStruct((btot,), jnp.float32),
        scratch_types=[
            pltpu.VMEM((chunk,), jnp.int32),
            pltpu.VMEM((chunk,), jnp.float32),
            pltpu.SemaphoreType.DMA,
        ],
    )
    def gather_k(table_hbm, idx_hbm, out_hbm, idx_v, vals_v, sem):
        wid = lax.axis_index("s") * info.num_cores + lax.axis_index("c")
        base = wid * b_per_w
        for ci in range(nchunk):
            off = base + ci * chunk
            pltpu.sync_copy(idx_hbm.at[pl.ds(off, chunk)], idx_v)
            pltpu.async_copy(table_hbm.at[idx_v], vals_v, sem).wait()
            pltpu.sync_copy(vals_v, out_hbm.at[pl.ds(off, chunk)])

    return gather_k(table_flat, idx_flat)


def kernel(points_sdf, W1, b1, W2, b2, bias):
    B, N, _ = points_sdf.shape
    centers = points_sdf[:, :CENTER_N, :]

    sel = functools.partial(_select_kernel, n_points=N, k_sel=FILTER_K)
    idx = pl.pallas_call(
        sel,
        grid=(B, CENTER_N // CB),
        in_specs=[
            pl.BlockSpec((1, N, 4), lambda b, j: (b, 0, 0)),
            pl.BlockSpec((1, CB, 4), lambda b, j: (b, j, 0)),
        ],
        out_specs=pl.BlockSpec((1, CB, FILTER_K * ROW_D),
                               lambda b, j: (b, j, 0)),
        out_shape=jax.ShapeDtypeStruct((B, CENTER_N, FILTER_K * ROW_D),
                                       jnp.int32),
    )(points_sdf, centers)

    gathered = _sc_gather(points_sdf.reshape(-1), idx.reshape(-1))
    gathered = gathered.reshape(B, CENTER_N * FILTER_K, ROW_D)

    mlp = functools.partial(_mlp_kernel, k_sel=FILTER_K)
    out = pl.pallas_call(
        mlp,
        grid=(B, CENTER_N // CB2),
        in_specs=[
            pl.BlockSpec((1, CB2 * FILTER_K, ROW_D), lambda b, j: (b, j, 0)),
            pl.BlockSpec((1, CB2, 4), lambda b, j: (b, j, 0)),
            pl.BlockSpec((3, HIDDEN), lambda b, j: (0, 0)),
            pl.BlockSpec((1, HIDDEN), lambda b, j: (0, 0)),
            pl.BlockSpec((HIDDEN, OUT_CH), lambda b, j: (0, 0)),
            pl.BlockSpec((1, OUT_CH), lambda b, j: (0, 0)),
            pl.BlockSpec((1, OUT_CH), lambda b, j: (0, 0)),
        ],
        out_specs=pl.BlockSpec((1, CB2, OUT_CH), lambda b, j: (b, j, 0)),
        out_shape=jax.ShapeDtypeStruct((B, CENTER_N, OUT_CH), jnp.float32),
    )(gathered, centers, W1, b1[None, :], W2, b2[None, :], bias[None, :])
    return out


# speedup vs baseline: 7.2718x; 2.5885x over previous
import functools

import jax
import jax.numpy as jnp
from jax import lax
from jax.experimental import pallas as pl
from jax.experimental.pallas import tpu as pltpu
from jax.experimental.pallas import tpu_sc as plsc

EDGE_LENGTH = 0.01
FILTER_K = 32
CENTER_N = 4096
OUT_CH = 32
HIDDEN = 16
ROW_D = 4
CHUNK_W = 8

CB = 256
CB3 = 1024
CB2 = 512


def _dist_prune_kernel(pts_ref, ctr_ref, d2_ref, cidx_ref, n_ref,
                       *, n_points, k_sel):
    pts = pts_ref[0]
    coords = pts[:, :3]
    ccoords = ctr_ref[0][:, :3]
    nrows = ccoords.shape[0]
    nchunk = n_points // CHUNK_W

    c2 = jnp.sum(ccoords * ccoords, axis=1, keepdims=True)
    p2 = jnp.sum(coords * coords, axis=1)[None, :]
    dot = lax.dot_general(
        ccoords, coords, (((1,), (1,)), ((), ())),
        preferred_element_type=jnp.float32)
    d2 = c2 + p2 - 2.0 * dot
    d2_ref[0] = d2

    cm = d2[:, :nchunk]
    for j in range(1, CHUNK_W):
        cm = jnp.minimum(cm, d2[:, j * nchunk:(j + 1) * nchunk])

    ciota = lax.broadcasted_iota(jnp.int32, (nrows, nchunk), 1)
    klane = lax.broadcasted_iota(jnp.int32, (nrows, k_sel), 1)

    def body(k, carry):
        m, i, cids = carry
        live = (cm > m) | ((cm == m) & (ciota > i))
        vm = jnp.where(live, cm, jnp.inf)
        m2 = jnp.min(vm, axis=1, keepdims=True)
        i2 = jnp.min(jnp.where(vm == m2, ciota, nchunk),
                     axis=1, keepdims=True)
        cids = jnp.where(klane == k, i2, cids)
        return m2, i2, cids

    m0 = jnp.full((nrows, 1), -jnp.inf, dtype=jnp.float32)
    i0 = jnp.full((nrows, 1), -1, dtype=jnp.int32)
    c0 = jnp.zeros((nrows, k_sel), dtype=jnp.int32)
    _, _, cids = lax.fori_loop(0, k_sel, body, (m0, i0, c0))

    cand_n = jnp.concatenate(
        [cids + j * nchunk for j in range(CHUNK_W)], axis=1)
    row = lax.broadcasted_iota(jnp.int32, (nrows, 1), 0)
    gid = (pl.program_id(0) * (CENTER_N // nrows) * nrows
           + pl.program_id(1) * nrows + row)
    cidx_ref[0] = gid * n_points + cand_n
    n_ref[0] = cand_n


def _topk_kernel(v_ref, n_ref, idx_ref, *, n_points, k_sel):
    v = v_ref[0]
    narr = n_ref[0]
    nrows = v.shape[0]
    lane = lax.broadcasted_iota(jnp.int32, (nrows, k_sel * ROW_D), 1)
    base = pl.program_id(0) * n_points

    def body(k, carry):
        m, i, idxs = carry
        live = (v > m) | ((v == m) & (narr > i))
        vm = jnp.where(live, v, jnp.inf)
        m2 = jnp.min(vm, axis=1, keepdims=True)
        n2 = jnp.min(jnp.where(vm == m2, narr, n_points),
                     axis=1, keepdims=True)
        elem = (n2 + base) * ROW_D + (lane & (ROW_D - 1))
        idxs = jnp.where((lane // ROW_D) == k, elem, idxs)
        return m2, n2, idxs

    m0 = jnp.full((nrows, 1), -jnp.inf, dtype=jnp.float32)
    i0 = jnp.full((nrows, 1), -1, dtype=jnp.int32)
    idx0 = jnp.zeros((nrows, k_sel * ROW_D), dtype=jnp.int32)
    _, _, idxs = lax.fori_loop(0, k_sel, body, (m0, i0, idx0))
    idx_ref[0] = idxs


def _mlp_kernel(nbr_ref, ctr_ref, W1_ref, b1_ref, W2_ref, b2_ref, bias_ref,
                out_ref, *, k_sel):
    nrows = ctr_ref.shape[1]
    nbr = nbr_ref[0]
    ctr = ctr_ref[0][:, :3]

    nbr3 = nbr[:, :3].reshape(nrows, k_sel, 3)
    rel = (nbr3 - ctr[:, None, :]) / EDGE_LENGTH
    rel = rel.reshape(nrows * k_sel, 3)
    h = jax.nn.relu(
        lax.dot_general(rel.astype(jnp.bfloat16),
                        W1_ref[...].astype(jnp.bfloat16),
                        (((1,), (0,)), ((), ())),
                        preferred_element_type=jnp.float32) + b1_ref[0])
    w = lax.dot_general(
        h.astype(jnp.bfloat16), W2_ref[...].astype(jnp.bfloat16),
        (((1,), (0,)), ((), ())),
        preferred_element_type=jnp.float32) + b2_ref[0]
    w = w * nbr[:, 3:4]
    acc = jnp.sum(w.reshape(nrows, k_sel, OUT_CH), axis=1)
    out_ref[0] = acc / k_sel + bias_ref[0]


def _sc_gather(table_flat, idx_flat):
    info = plsc.get_sparse_core_info()
    nw = info.num_cores * info.num_subcores
    btot = idx_flat.shape[0]
    b_per_w = btot // nw
    chunk = min(b_per_w, 4096)
    nchunk = b_per_w // chunk
    mesh = plsc.VectorSubcoreMesh(core_axis_name="c", subcore_axis_name="s")

    @functools.partial(
        pl.kernel, mesh=mesh,
        out_type=jax.ShapeDtypeStruct((btot,), jnp.float32),
        scratch_types=[
            pltpu.VMEM((chunk,), jnp.int32),
            pltpu.VMEM((chunk,), jnp.float32),
            pltpu.SemaphoreType.DMA,
        ],
    )
    def gather_k(table_hbm, idx_hbm, out_hbm, idx_v, vals_v, sem):
        wid = lax.axis_index("s") * info.num_cores + lax.axis_index("c")
        base = wid * b_per_w
        for ci in range(nchunk):
            off = base + ci * chunk
            pltpu.sync_copy(idx_hbm.at[pl.ds(off, chunk)], idx_v)
            pltpu.async_copy(table_hbm.at[idx_v], vals_v, sem).wait()
            pltpu.sync_copy(vals_v, out_hbm.at[pl.ds(off, chunk)])

    return gather_k(table_flat, idx_flat)


def kernel(points_sdf, W1, b1, W2, b2, bias):
    B, N, _ = points_sdf.shape
    centers = points_sdf[:, :CENTER_N, :]
    ncand = CHUNK_W * FILTER_K

    dp = functools.partial(_dist_prune_kernel, n_points=N, k_sel=FILTER_K)
    d2_full, cand_gidx, cand_n = pl.pallas_call(
        dp,
        grid=(B, CENTER_N // CB),
        in_specs=[
            pl.BlockSpec((1, N, 4), lambda b, j: (b, 0, 0)),
            pl.BlockSpec((1, CB, 4), lambda b, j: (b, j, 0)),
        ],
        out_specs=[
            pl.BlockSpec((1, CB, N), lambda b, j: (b, j, 0)),
            pl.BlockSpec((1, CB, ncand), lambda b, j: (b, j, 0)),
            pl.BlockSpec((1, CB, ncand), lambda b, j: (b, j, 0)),
        ],
        out_shape=[
            jax.ShapeDtypeStruct((B, CENTER_N, N), jnp.float32),
            jax.ShapeDtypeStruct((B, CENTER_N, ncand), jnp.int32),
            jax.ShapeDtypeStruct((B, CENTER_N, ncand), jnp.int32),
        ],
    )(points_sdf, centers)

    cand_vals = _sc_gather(d2_full.reshape(-1), cand_gidx.reshape(-1))
    cand_vals = cand_vals.reshape(B, CENTER_N, ncand)

    tk = functools.partial(_topk_kernel, n_points=N, k_sel=FILTER_K)
    idx = pl.pallas_call(
        tk,
        grid=(B, CENTER_N // CB3),
        in_specs=[
            pl.BlockSpec((1, CB3, ncand), lambda b, j: (b, j, 0)),
            pl.BlockSpec((1, CB3, ncand), lambda b, j: (b, j, 0)),
        ],
        out_specs=pl.BlockSpec((1, CB3, FILTER_K * ROW_D),
                               lambda b, j: (b, j, 0)),
        out_shape=jax.ShapeDtypeStruct((B, CENTER_N, FILTER_K * ROW_D),
                                       jnp.int32),
    )(cand_vals, cand_n)

    gathered = _sc_gather(points_sdf.reshape(-1), idx.reshape(-1))
    gathered = gathered.reshape(B, CENTER_N * FILTER_K, ROW_D)

    mlp = functools.partial(_mlp_kernel, k_sel=FILTER_K)
    out = pl.pallas_call(
        mlp,
        grid=(B, CENTER_N // CB2),
        in_specs=[
            pl.BlockSpec((1, CB2 * FILTER_K, ROW_D), lambda b, j: (b, j, 0)),
            pl.BlockSpec((1, CB2, 4), lambda b, j: (b, j, 0)),
            pl.BlockSpec((3, HIDDEN), lambda b, j: (0, 0)),
            pl.BlockSpec((1, HIDDEN), lambda b, j: (0, 0)),
            pl.BlockSpec((HIDDEN, OUT_CH), lambda b, j: (0, 0)),
            pl.BlockSpec((1, OUT_CH), lambda b, j: (0, 0)),
            pl.BlockSpec((1, OUT_CH), lambda b, j: (0, 0)),
        ],
        out_specs=pl.BlockSpec((1, CB2, OUT_CH), lambda b, j: (b, j, 0)),
        out_shape=jax.ShapeDtypeStruct((B, CENTER_N, OUT_CH), jnp.float32),
    )(gathered, centers, W1, b1[None, :], W2, b2[None, :], bias[None, :])
    return out


# speedup vs baseline: 8.1616x; 1.1224x over previous
import functools

import jax
import jax.numpy as jnp
from jax import lax
from jax.experimental import pallas as pl
from jax.experimental.pallas import tpu as pltpu
from jax.experimental.pallas import tpu_sc as plsc

EDGE_LENGTH = 0.01
FILTER_K = 32
CENTER_N = 4096
OUT_CH = 32
HIDDEN = 16
ROW_D = 4
CHUNK_W = 8

CB = 256
CB3 = 1024
CB2 = 512


def _dist_prune_kernel(pts_ref, ctr_ref, d2_ref, cidx_ref, n_ref, cm_ref,
                       *, n_points, k_sel):
    pts = pts_ref[0]
    coords = pts[:, :3]
    ccoords = ctr_ref[0][:, :3]
    nrows = ccoords.shape[0]
    nchunk = n_points // CHUNK_W

    c2 = jnp.sum(ccoords * ccoords, axis=1, keepdims=True)
    p2 = jnp.sum(coords * coords, axis=1)[None, :]
    dot = lax.dot_general(
        ccoords, coords, (((1,), (1,)), ((), ())),
        preferred_element_type=jnp.float32)
    d2 = c2 + p2 - 2.0 * dot
    d2_ref[0] = d2

    cm = d2[:, :nchunk]
    for j in range(1, CHUNK_W):
        cm = jnp.minimum(cm, d2[:, j * nchunk:(j + 1) * nchunk])
    cm_ref[...] = cm

    ciota = lax.broadcasted_iota(jnp.int32, (nrows, nchunk), 1)
    klane = lax.broadcasted_iota(jnp.int32, (nrows, k_sel), 1)

    def body(k, carry):
        i, cids = carry
        c = jnp.where(ciota == i, jnp.inf, cm_ref[...])
        cm_ref[...] = c
        m2 = jnp.min(c, axis=1, keepdims=True)
        i2 = jnp.min(jnp.where(c == m2, ciota, nchunk),
                     axis=1, keepdims=True)
        cids = jnp.where(klane == k, i2, cids)
        return i2, cids

    i0 = jnp.full((nrows, 1), -1, dtype=jnp.int32)
    c0 = jnp.zeros((nrows, k_sel), dtype=jnp.int32)
    _, cids = lax.fori_loop(0, k_sel, body, (i0, c0))

    cand_n = jnp.concatenate(
        [cids + j * nchunk for j in range(CHUNK_W)], axis=1)
    row = lax.broadcasted_iota(jnp.int32, (nrows, 1), 0)
    gid = (pl.program_id(0) * (CENTER_N // nrows) * nrows
           + pl.program_id(1) * nrows + row)
    cidx_ref[0] = gid * n_points + cand_n
    n_ref[0] = cand_n


def _topk_kernel(v_ref, n_ref, idx_ref, *, n_points, k_sel):
    v = v_ref[0]
    narr = n_ref[0]
    nrows = v.shape[0]
    lane = lax.broadcasted_iota(jnp.int32, (nrows, k_sel * ROW_D), 1)
    base = pl.program_id(0) * n_points

    def body(k, carry):
        m, i, idxs = carry
        live = (v > m) | ((v == m) & (narr > i))
        vm = jnp.where(live, v, jnp.inf)
        m2 = jnp.min(vm, axis=1, keepdims=True)
        n2 = jnp.min(jnp.where(vm == m2, narr, n_points),
                     axis=1, keepdims=True)
        elem = (n2 + base) * ROW_D + (lane & (ROW_D - 1))
        idxs = jnp.where((lane // ROW_D) == k, elem, idxs)
        return m2, n2, idxs

    m0 = jnp.full((nrows, 1), -jnp.inf, dtype=jnp.float32)
    i0 = jnp.full((nrows, 1), -1, dtype=jnp.int32)
    idx0 = jnp.zeros((nrows, k_sel * ROW_D), dtype=jnp.int32)
    _, _, idxs = lax.fori_loop(0, k_sel, body, (m0, i0, idx0))
    idx_ref[0] = idxs


def _mlp_kernel(nbr_ref, ctr_ref, W1_ref, b1_ref, W2_ref, b2_ref, bias_ref,
                out_ref, *, k_sel):
    nrows = ctr_ref.shape[1]
    nbr = nbr_ref[0]
    ctr = ctr_ref[0][:, :3]

    nbr3 = nbr[:, :3].reshape(nrows, k_sel, 3)
    rel = (nbr3 - ctr[:, None, :]) / EDGE_LENGTH
    rel = rel.reshape(nrows * k_sel, 3)
    h = jax.nn.relu(
        lax.dot_general(rel.astype(jnp.bfloat16),
                        W1_ref[...].astype(jnp.bfloat16),
                        (((1,), (0,)), ((), ())),
                        preferred_element_type=jnp.float32) + b1_ref[0])
    w = lax.dot_general(
        h.astype(jnp.bfloat16), W2_ref[...].astype(jnp.bfloat16),
        (((1,), (0,)), ((), ())),
        preferred_element_type=jnp.float32) + b2_ref[0]
    w = w * nbr[:, 3:4]
    acc = jnp.sum(w.reshape(nrows, k_sel, OUT_CH), axis=1)
    out_ref[0] = acc / k_sel + bias_ref[0]


def _sc_gather(table_flat, idx_flat):
    info = plsc.get_sparse_core_info()
    nw = info.num_cores * info.num_subcores
    btot = idx_flat.shape[0]
    b_per_w = btot // nw
    chunk = min(b_per_w, 4096)
    nchunk = b_per_w // chunk
    mesh = plsc.VectorSubcoreMesh(core_axis_name="c", subcore_axis_name="s")

    @functools.partial(
        pl.kernel, mesh=mesh,
        out_type=jax.ShapeDtypeStruct((btot,), jnp.float32),
        scratch_types=[
            pltpu.VMEM((chunk,), jnp.int32),
            pltpu.VMEM((chunk,), jnp.float32),
            pltpu.SemaphoreType.DMA,
        ],
    )
    def gather_k(table_hbm, idx_hbm, out_hbm, idx_v, vals_v, sem):
        wid = lax.axis_index("s") * info.num_cores + lax.axis_index("c")
        base = wid * b_per_w
        for ci in range(nchunk):
            off = base + ci * chunk
            pltpu.sync_copy(idx_hbm.at[pl.ds(off, chunk)], idx_v)
            pltpu.async_copy(table_hbm.at[idx_v], vals_v, sem).wait()
            pltpu.sync_copy(vals_v, out_hbm.at[pl.ds(off, chunk)])

    return gather_k(table_flat, idx_flat)


def kernel(points_sdf, W1, b1, W2, b2, bias):
    B, N, _ = points_sdf.shape
    centers = points_sdf[:, :CENTER_N, :]
    ncand = CHUNK_W * FILTER_K

    dp = functools.partial(_dist_prune_kernel, n_points=N, k_sel=FILTER_K)
    d2_full, cand_gidx, cand_n = pl.pallas_call(
        dp,
        grid=(B, CENTER_N // CB),
        in_specs=[
            pl.BlockSpec((1, N, 4), lambda b, j: (b, 0, 0)),
            pl.BlockSpec((1, CB, 4), lambda b, j: (b, j, 0)),
        ],
        out_specs=[
            pl.BlockSpec((1, CB, N), lambda b, j: (b, j, 0)),
            pl.BlockSpec((1, CB, ncand), lambda b, j: (b, j, 0)),
            pl.BlockSpec((1, CB, ncand), lambda b, j: (b, j, 0)),
        ],
        out_shape=[
            jax.ShapeDtypeStruct((B, CENTER_N, N), jnp.float32),
            jax.ShapeDtypeStruct((B, CENTER_N, ncand), jnp.int32),
            jax.ShapeDtypeStruct((B, CENTER_N, ncand), jnp.int32),
        ],
        scratch_shapes=[pltpu.VMEM((CB, N // CHUNK_W), jnp.float32)],
    )(points_sdf, centers)

    cand_vals = _sc_gather(d2_full.reshape(-1), cand_gidx.reshape(-1))
    cand_vals = cand_vals.reshape(B, CENTER_N, ncand)

    tk = functools.partial(_topk_kernel, n_points=N, k_sel=FILTER_K)
    idx = pl.pallas_call(
        tk,
        grid=(B, CENTER_N // CB3),
        in_specs=[
            pl.BlockSpec((1, CB3, ncand), lambda b, j: (b, j, 0)),
            pl.BlockSpec((1, CB3, ncand), lambda b, j: (b, j, 0)),
        ],
        out_specs=pl.BlockSpec((1, CB3, FILTER_K * ROW_D),
                               lambda b, j: (b, j, 0)),
        out_shape=jax.ShapeDtypeStruct((B, CENTER_N, FILTER_K * ROW_D),
                                       jnp.int32),
    )(cand_vals, cand_n)

    gathered = _sc_gather(points_sdf.reshape(-1), idx.reshape(-1))
    gathered = gathered.reshape(B, CENTER_N * FILTER_K, ROW_D)

    mlp = functools.partial(_mlp_kernel, k_sel=FILTER_K)
    out = pl.pallas_call(
        mlp,
        grid=(B, CENTER_N // CB2),
        in_specs=[
            pl.BlockSpec((1, CB2 * FILTER_K, ROW_D), lambda b, j: (b, j, 0)),
            pl.BlockSpec((1, CB2, 4), lambda b, j: (b, j, 0)),
            pl.BlockSpec((3, HIDDEN), lambda b, j: (0, 0)),
            pl.BlockSpec((1, HIDDEN), lambda b, j: (0, 0)),
            pl.BlockSpec((HIDDEN, OUT_CH), lambda b, j: (0, 0)),
            pl.BlockSpec((1, OUT_CH), lambda b, j: (0, 0)),
            pl.BlockSpec((1, OUT_CH), lambda b, j: (0, 0)),
        ],
        out_specs=pl.BlockSpec((1, CB2, OUT_CH), lambda b, j: (b, j, 0)),
        out_shape=jax.ShapeDtypeStruct((B, CENTER_N, OUT_CH), jnp.float32),
    )(gathered, centers, W1, b1[None, :], W2, b2[None, :], bias[None, :])
    return out
